# baseline (speedup 1.0000x reference)
import functools

import jax
import jax.numpy as jnp
from jax import lax
from jax.experimental import pallas as pl
from jax.experimental.pallas import tpu as pltpu
from jax.experimental.pallas import tpu_sc as plsc

N = 10000
N_PAD = 10112
D = 256
HALF = 128
E = 160000
G = 16
O = 10

NS = 16
E_PAD = 163840
EDGES_PER_TILE = E_PAD // NS
CHUNK = 40
N_CHUNKS = EDGES_PER_TILE // CHUNK
ROWS_PER_TILE = N_PAD // NS
ACC_ROWS = N_PAD

RB = 1264
NB = N_PAD // RB


K_SLOTS = 4
N_BUFS = 2 * K_SLOTS
N_ISLOT = 4
N_WAVES = N_CHUNKS // K_SLOTS


@functools.partial(
    pl.kernel,
    out_type=(
        jax.ShapeDtypeStruct((N_PAD, HALF), jnp.float32),
        jax.ShapeDtypeStruct((N_PAD, HALF), jnp.float32),
    ),
    mesh=plsc.VectorSubcoreMesh(core_axis_name="c", subcore_axis_name="s"),
    scratch_types=[
        pltpu.VMEM_SHARED((ACC_ROWS, HALF), jnp.float32),
        pltpu.VMEM((N_ISLOT, K_SLOTS, CHUNK), jnp.int32),
        pltpu.VMEM((N_ISLOT, K_SLOTS, CHUNK), jnp.int32),
    ] + [pltpu.VMEM((CHUNK, HALF), jnp.float32)] * N_BUFS
      + [pltpu.SemaphoreType.DMA] * (2 * N_BUFS + 1),
)
def _sc_segsum(h_lo, h_hi, src4, dst4, z_lo, z_hi, acc, sidx, didx, *rest):
    bufs = rest[:N_BUFS]
    gsem = rest[N_BUFS:2 * N_BUFS]
    ssem = rest[2 * N_BUFS:3 * N_BUFS]
    isem = rest[3 * N_BUFS]
    c = lax.axis_index("c")
    s = lax.axis_index("s")
    row0 = s * ROWS_PER_TILE

    halves = ((0, h_lo, z_lo), (1, h_hi, z_hi))

    for cc, h_ref, _ in halves:
        @pl.when(c == cc)
        def _():
            pltpu.sync_copy(h_ref.at[pl.ds(row0, ROWS_PER_TILE)],
                            acc.at[pl.ds(row0, ROWS_PER_TILE)])
    plsc.subcore_barrier()

    for cc, h_ref, _ in halves:
        @pl.when(c == cc)
        def _():
            pltpu.sync_copy(src4.at[s, 0], sidx.at[0])
            pltpu.sync_copy(dst4.at[s, 0], didx.at[0])
            pltpu.async_copy(src4.at[s, 1], sidx.at[1], isem)
            pltpu.async_copy(dst4.at[s, 1], didx.at[1], isem)
            for k in range(K_SLOTS):
                pltpu.async_copy(h_ref.at[sidx.at[0, k]], bufs[k], gsem[k])

            def quad(q, carry):
                for u in range(4):
                    w = 4 * q + u
                    pb = u % 2
                    ob = 1 - pb
                    for k in range(K_SLOTS):
                        @pl.when(w >= 2)
                        def _():
                            pltpu.make_async_copy(
                                bufs[pb * K_SLOTS + k],
                                acc.at[didx.at[0, 0]],
                                ssem[pb * K_SLOTS + k]).wait()
                    @pl.when(w + 1 < N_WAVES)
                    def _():
                        pltpu.make_async_copy(
                            src4.at[s, 0], sidx.at[0], isem).wait()
                        pltpu.make_async_copy(
                            dst4.at[s, 0], didx.at[0], isem).wait()
                    for k in range(K_SLOTS):
                        @pl.when(w + 1 < N_WAVES)
                        def _():
                            pltpu.async_copy(
                                h_ref.at[sidx.at[(u + 1) % N_ISLOT, k]],
                                bufs[ob * K_SLOTS + k],
                                gsem[ob * K_SLOTS + k])
                    @pl.when(w + 2 < N_WAVES)
                    def _():
                        pltpu.async_copy(src4.at[s, w + 2],
                                         sidx.at[(u + 2) % N_ISLOT], isem)
                        pltpu.async_copy(dst4.at[s, w + 2],
                                         didx.at[(u + 2) % N_ISLOT], isem)
                    for k in range(K_SLOTS):
                        pltpu.make_async_copy(
                            h_ref.at[sidx.at[0, 0]],
                            bufs[pb * K_SLOTS + k],
                            gsem[pb * K_SLOTS + k]).wait()
                        pltpu.async_copy(bufs[pb * K_SLOTS + k],
                                         acc.at[didx.at[u % N_ISLOT, k]],
                                         ssem[pb * K_SLOTS + k], add=True)
                return carry

            lax.fori_loop(0, N_WAVES // 4, quad, 0)
            for k in range(N_BUFS):
                pltpu.make_async_copy(bufs[k], acc.at[didx.at[0, 0]],
                                      ssem[k]).wait()
    plsc.subcore_barrier()

    for cc, _, z_ref in halves:
        @pl.when(c == cc)
        def _():
            pltpu.sync_copy(acc.at[pl.ds(row0, ROWS_PER_TILE)],
                            z_ref.at[pl.ds(row0, ROWS_PER_TILE)])


def _tc_layer_body(zl, zh, W1, b1, W2, b2, g, bln, ol, oh):
    z = jnp.concatenate([zl[...], zh[...]], axis=1)
    a = jnp.dot(z, W1[...], preferred_element_type=jnp.float32) + b1[...]
    a = jnp.maximum(a, 0.0)
    y = jnp.dot(a, W2[...], preferred_element_type=jnp.float32) + b2[...]
    y = jnp.maximum(y, 0.0)
    m = jnp.mean(y, axis=1, keepdims=True)
    v = jnp.mean((y - m) * (y - m), axis=1, keepdims=True)
    y = (y - m) * lax.rsqrt(v + 1e-5) * g[...] + bln[...]
    ol[...] = y[:, :HALF]
    oh[...] = y[:, HALF:]


def _tc_layer(zl, zh, W1, b1, W2, b2, g, bln):
    full = pl.BlockSpec((D, D), lambda i: (0, 0))
    vec = pl.BlockSpec((1, D), lambda i: (0, 0))
    half = pl.BlockSpec((RB, HALF), lambda i: (i, 0))
    return pl.pallas_call(
        _tc_layer_body,
        grid=(NB,),
        in_specs=[half, half, full, vec, full, vec, vec, vec],
        out_specs=[half, half],
        out_shape=(
            jax.ShapeDtypeStruct((N_PAD, HALF), jnp.float32),
            jax.ShapeDtypeStruct((N_PAD, HALF), jnp.float32),
        ),
    )(zl, zh, W1, b1, W2, b2, g, bln)


def _tc_final_body(zl, zh, batch_r, W1, b1, W2, b2, pW1, pb1, pW2, pb2,
                   emb_ref, out_ref, sums, cnts):
    i = pl.program_id(0)
    z = jnp.concatenate([zl[...], zh[...]], axis=1)
    a = jnp.dot(z, W1[...], preferred_element_type=jnp.float32) + b1[...]
    a = jnp.maximum(a, 0.0)
    y = jnp.dot(a, W2[...], preferred_element_type=jnp.float32) + b2[...]
    y = jnp.maximum(y, 0.0)

    b = batch_r[0, 0, :].reshape(RB, 1)
    gids = lax.broadcasted_iota(jnp.int32, (RB, G), 1)
    onehot = (b == gids).astype(jnp.float32)
    part = lax.dot_general(onehot, y, (((0,), (0,)), ((), ())),
                           preferred_element_type=jnp.float32)
    ones = jnp.ones((RB, HALF), jnp.float32)
    pcnt = lax.dot_general(onehot, ones, (((0,), (0,)), ((), ())),
                           preferred_element_type=jnp.float32)

    @pl.when(i == 0)
    def _():
        sums[...] = part
        cnts[...] = pcnt

    @pl.when(i > 0)
    def _():
        sums[...] = sums[...] + part
        cnts[...] = cnts[...] + pcnt

    @pl.when(i == NB - 1)
    def _():
        cnt = jnp.maximum(cnts[...][:, 0:1], 1.0)
        emb = sums[...] / cnt
        emb_ref[...] = emb
        z1 = jnp.dot(emb, pW1[...], preferred_element_type=jnp.float32) + pb1[...]
        z2 = jnp.dot(z1, pW2[...], preferred_element_type=jnp.float32) + pb2[...]
        mask = lax.broadcasted_iota(jnp.int32, (G, HALF), 1) < O
        zm = jnp.where(mask, z2, -1e30)
        mx = jnp.max(zm, axis=1, keepdims=True)
        se = jnp.sum(jnp.where(mask, jnp.exp(z2 - mx), 0.0), axis=1, keepdims=True)
        out_ref[...] = z2 - (jnp.log(se) + mx)


def _tc_final(zl, zh, batch_r, W1, b1, W2, b2, pW1, pb1, pW2, pb2):
    full = pl.BlockSpec((D, D), lambda i: (0, 0))
    vec = pl.BlockSpec((1, D), lambda i: (0, 0))
    half = pl.BlockSpec((RB, HALF), lambda i: (i, 0))
    return pl.pallas_call(
        _tc_final_body,
        grid=(NB,),
        in_specs=[half, half,
                  pl.BlockSpec((1, 1, RB), lambda i: (i, 0, 0)),
                  full, vec, full, vec,
                  full, vec,
                  pl.BlockSpec((D, HALF), lambda i: (0, 0)),
                  pl.BlockSpec((1, HALF), lambda i: (0, 0))],
        out_specs=[pl.BlockSpec((G, D), lambda i: (0, 0)),
                   pl.BlockSpec((G, HALF), lambda i: (0, 0))],
        out_shape=(
            jax.ShapeDtypeStruct((G, D), jnp.float32),
            jax.ShapeDtypeStruct((G, HALF), jnp.float32),
        ),
        scratch_shapes=[pltpu.VMEM((G, D), jnp.float32),
                        pltpu.VMEM((G, HALF), jnp.float32)],
    )(zl, zh, batch_r, W1, b1, W2, b2, pW1, pb1, pW2, pb2)


def kernel(x, edge_index, batch,
           c0_W1, c0_b1, c0_W2, c0_b2,
           c1_W1, c1_b1, c1_W2, c1_b2,
           c2_W1, c2_b1, c2_W2, c2_b2,
           ln0_g, ln0_b, ln1_g, ln1_b,
           p_W1, p_b1, p_W2, p_b2):
    src = jnp.concatenate([edge_index[0], jnp.zeros((E_PAD - E,), jnp.int32)])
    dst = jnp.concatenate([edge_index[1], jnp.full((E_PAD - E,), N, jnp.int32)])
    src = src.reshape(NS, N_WAVES, K_SLOTS, CHUNK)
    dst = dst.reshape(NS, N_WAVES, K_SLOTS, CHUNK)

    row = lambda v: v.reshape(1, -1)
    pW2 = jnp.pad(p_W2, ((0, 0), (0, HALF - O)))
    pb2 = jnp.pad(p_b2, (0, HALF - O)).reshape(1, HALF)
    batch_r = jnp.pad(batch, (0, N_PAD - N), constant_values=G).reshape(NB, 1, RB)

    xp = jnp.pad(x, ((0, N_PAD - N), (0, 0)))
    h_lo, h_hi = xp[:, :HALF], xp[:, HALF:]
    z_lo, z_hi = _sc_segsum(h_lo, h_hi, src, dst)
    h_lo, h_hi = _tc_layer(z_lo, z_hi, c0_W1, row(c0_b1), c0_W2, row(c0_b2),
                           row(ln0_g), row(ln0_b))
    z_lo, z_hi = _sc_segsum(h_lo, h_hi, src, dst)
    h_lo, h_hi = _tc_layer(z_lo, z_hi, c1_W1, row(c1_b1), c1_W2, row(c1_b2),
                           row(ln1_g), row(ln1_b))
    z_lo, z_hi = _sc_segsum(h_lo, h_hi, src, dst)
    emb, out_pad = _tc_final(z_lo, z_hi, batch_r,
                             c2_W1, row(c2_b1), c2_W2, row(c2_b2),
                             p_W1, row(p_b1), pW2, pb2)
    return (emb, out_pad[:, :O])


# speedup vs baseline: 1.0401x; 1.0401x over previous
import functools

import jax
import jax.numpy as jnp
from jax import lax
from jax.experimental import pallas as pl
from jax.experimental.pallas import tpu as pltpu
from jax.experimental.pallas import tpu_sc as plsc

N = 10000
N_PAD = 10112
D = 256
HALF = 128
E = 160000
G = 16
O = 10

NS = 16
E_PAD = 163840
EDGES_PER_TILE = E_PAD // NS
CHUNK = 80
N_CHUNKS = EDGES_PER_TILE // CHUNK
ROWS_PER_TILE = N_PAD // NS
ACC_ROWS = N_PAD

RB = 1264
NB = N_PAD // RB


K_SLOTS = 2
N_BUFS = 2 * K_SLOTS
N_ISLOT = 4
N_WAVES = N_CHUNKS // K_SLOTS


@functools.partial(
    pl.kernel,
    out_type=(
        jax.ShapeDtypeStruct((N_PAD, HALF), jnp.float32),
        jax.ShapeDtypeStruct((N_PAD, HALF), jnp.float32),
    ),
    mesh=plsc.VectorSubcoreMesh(core_axis_name="c", subcore_axis_name="s"),
    scratch_types=[
        pltpu.VMEM_SHARED((ACC_ROWS, HALF), jnp.float32),
        pltpu.VMEM((N_ISLOT, K_SLOTS, CHUNK), jnp.int32),
        pltpu.VMEM((N_ISLOT, K_SLOTS, CHUNK), jnp.int32),
    ] + [pltpu.VMEM((CHUNK, HALF), jnp.float32)] * N_BUFS
      + [pltpu.SemaphoreType.DMA] * (2 * N_BUFS + 1),
)
def _sc_segsum(h_lo, h_hi, src4, dst4, z_lo, z_hi, acc, sidx, didx, *rest):
    bufs = rest[:N_BUFS]
    gsem = rest[N_BUFS:2 * N_BUFS]
    ssem = rest[2 * N_BUFS:3 * N_BUFS]
    isem = rest[3 * N_BUFS]
    c = lax.axis_index("c")
    s = lax.axis_index("s")
    row0 = s * ROWS_PER_TILE

    halves = ((0, h_lo, z_lo), (1, h_hi, z_hi))

    for cc, h_ref, _ in halves:
        @pl.when(c == cc)
        def _():
            pltpu.sync_copy(src4.at[s, 0], sidx.at[0])
            pltpu.sync_copy(dst4.at[s, 0], didx.at[0])
            pltpu.async_copy(src4.at[s, 1], sidx.at[1], isem)
            pltpu.async_copy(dst4.at[s, 1], didx.at[1], isem)
            for k in range(K_SLOTS):
                pltpu.async_copy(h_ref.at[sidx.at[0, k]], bufs[k], gsem[k])
            pltpu.sync_copy(h_ref.at[pl.ds(row0, ROWS_PER_TILE)],
                            acc.at[pl.ds(row0, ROWS_PER_TILE)])
    plsc.subcore_barrier()

    for cc, h_ref, _ in halves:
        @pl.when(c == cc)
        def _():
            def quad(q, carry):
                for u in range(4):
                    w = 4 * q + u
                    pb = u % 2
                    ob = 1 - pb
                    for k in range(K_SLOTS):
                        @pl.when(w >= 2)
                        def _():
                            pltpu.make_async_copy(
                                bufs[pb * K_SLOTS + k],
                                acc.at[didx.at[0, 0]],
                                ssem[pb * K_SLOTS + k]).wait()
                    @pl.when(w + 1 < N_WAVES)
                    def _():
                        pltpu.make_async_copy(
                            src4.at[s, 0], sidx.at[0], isem).wait()
                        pltpu.make_async_copy(
                            dst4.at[s, 0], didx.at[0], isem).wait()
                    for k in range(K_SLOTS):
                        @pl.when(w + 1 < N_WAVES)
                        def _():
                            pltpu.async_copy(
                                h_ref.at[sidx.at[(u + 1) % N_ISLOT, k]],
                                bufs[ob * K_SLOTS + k],
                                gsem[ob * K_SLOTS + k])
                    @pl.when(w + 2 < N_WAVES)
                    def _():
                        pltpu.async_copy(src4.at[s, w + 2],
                                         sidx.at[(u + 2) % N_ISLOT], isem)
                        pltpu.async_copy(dst4.at[s, w + 2],
                                         didx.at[(u + 2) % N_ISLOT], isem)
                    for k in range(K_SLOTS):
                        pltpu.make_async_copy(
                            h_ref.at[sidx.at[0, 0]],
                            bufs[pb * K_SLOTS + k],
                            gsem[pb * K_SLOTS + k]).wait()
                        pltpu.async_copy(bufs[pb * K_SLOTS + k],
                                         acc.at[didx.at[u % N_ISLOT, k]],
                                         ssem[pb * K_SLOTS + k], add=True)
                return carry

            lax.fori_loop(0, N_WAVES // 4, quad, 0)
            for k in range(N_BUFS):
                pltpu.make_async_copy(bufs[k], acc.at[didx.at[0, 0]],
                                      ssem[k]).wait()
    plsc.subcore_barrier()

    for cc, _, z_ref in halves:
        @pl.when(c == cc)
        def _():
            pltpu.sync_copy(acc.at[pl.ds(row0, ROWS_PER_TILE)],
                            z_ref.at[pl.ds(row0, ROWS_PER_TILE)])


def _tc_layer_body(zl, zh, W1, b1, W2, b2, g, bln, ol, oh):
    z = jnp.concatenate([zl[...], zh[...]], axis=1)
    a = jnp.dot(z, W1[...], preferred_element_type=jnp.float32) + b1[...]
    a = jnp.maximum(a, 0.0)
    y = jnp.dot(a, W2[...], preferred_element_type=jnp.float32) + b2[...]
    y = jnp.maximum(y, 0.0)
    m = jnp.mean(y, axis=1, keepdims=True)
    v = jnp.mean((y - m) * (y - m), axis=1, keepdims=True)
    y = (y - m) * lax.rsqrt(v + 1e-5) * g[...] + bln[...]
    ol[...] = y[:, :HALF]
    oh[...] = y[:, HALF:]


def _tc_layer(zl, zh, W1, b1, W2, b2, g, bln):
    full = pl.BlockSpec((D, D), lambda i: (0, 0))
    vec = pl.BlockSpec((1, D), lambda i: (0, 0))
    half = pl.BlockSpec((RB, HALF), lambda i: (i, 0))
    return pl.pallas_call(
        _tc_layer_body,
        grid=(NB,),
        in_specs=[half, half, full, vec, full, vec, vec, vec],
        out_specs=[half, half],
        out_shape=(
            jax.ShapeDtypeStruct((N_PAD, HALF), jnp.float32),
            jax.ShapeDtypeStruct((N_PAD, HALF), jnp.float32),
        ),
    )(zl, zh, W1, b1, W2, b2, g, bln)


def _tc_final_body(zl, zh, batch_r, W1, b1, W2, b2, pW1, pb1, pW2, pb2,
                   emb_ref, out_ref, sums, cnts):
    i = pl.program_id(0)
    z = jnp.concatenate([zl[...], zh[...]], axis=1)
    a = jnp.dot(z, W1[...], preferred_element_type=jnp.float32) + b1[...]
    a = jnp.maximum(a, 0.0)
    y = jnp.dot(a, W2[...], preferred_element_type=jnp.float32) + b2[...]
    y = jnp.maximum(y, 0.0)

    b = batch_r[0, 0, :].reshape(RB, 1)
    gids = lax.broadcasted_iota(jnp.int32, (RB, G), 1)
    onehot = (b == gids).astype(jnp.float32)
    part = lax.dot_general(onehot, y, (((0,), (0,)), ((), ())),
                           preferred_element_type=jnp.float32)
    ones = jnp.ones((RB, HALF), jnp.float32)
    pcnt = lax.dot_general(onehot, ones, (((0,), (0,)), ((), ())),
                           preferred_element_type=jnp.float32)

    @pl.when(i == 0)
    def _():
        sums[...] = part
        cnts[...] = pcnt

    @pl.when(i > 0)
    def _():
        sums[...] = sums[...] + part
        cnts[...] = cnts[...] + pcnt

    @pl.when(i == NB - 1)
    def _():
        cnt = jnp.maximum(cnts[...][:, 0:1], 1.0)
        emb = sums[...] / cnt
        emb_ref[...] = emb
        z1 = jnp.dot(emb, pW1[...], preferred_element_type=jnp.float32) + pb1[...]
        z2 = jnp.dot(z1, pW2[...], preferred_element_type=jnp.float32) + pb2[...]
        mask = lax.broadcasted_iota(jnp.int32, (G, HALF), 1) < O
        zm = jnp.where(mask, z2, -1e30)
        mx = jnp.max(zm, axis=1, keepdims=True)
        se = jnp.sum(jnp.where(mask, jnp.exp(z2 - mx), 0.0), axis=1, keepdims=True)
        out_ref[...] = z2 - (jnp.log(se) + mx)


def _tc_final(zl, zh, batch_r, W1, b1, W2, b2, pW1, pb1, pW2, pb2):
    full = pl.BlockSpec((D, D), lambda i: (0, 0))
    vec = pl.BlockSpec((1, D), lambda i: (0, 0))
    half = pl.BlockSpec((RB, HALF), lambda i: (i, 0))
    return pl.pallas_call(
        _tc_final_body,
        grid=(NB,),
        in_specs=[half, half,
                  pl.BlockSpec((1, 1, RB), lambda i: (i, 0, 0)),
                  full, vec, full, vec,
                  full, vec,
                  pl.BlockSpec((D, HALF), lambda i: (0, 0)),
                  pl.BlockSpec((1, HALF), lambda i: (0, 0))],
        out_specs=[pl.BlockSpec((G, D), lambda i: (0, 0)),
                   pl.BlockSpec((G, HALF), lambda i: (0, 0))],
        out_shape=(
            jax.ShapeDtypeStruct((G, D), jnp.float32),
            jax.ShapeDtypeStruct((G, HALF), jnp.float32),
        ),
        scratch_shapes=[pltpu.VMEM((G, D), jnp.float32),
                        pltpu.VMEM((G, HALF), jnp.float32)],
    )(zl, zh, batch_r, W1, b1, W2, b2, pW1, pb1, pW2, pb2)


def kernel(x, edge_index, batch,
           c0_W1, c0_b1, c0_W2, c0_b2,
           c1_W1, c1_b1, c1_W2, c1_b2,
           c2_W1, c2_b1, c2_W2, c2_b2,
           ln0_g, ln0_b, ln1_g, ln1_b,
           p_W1, p_b1, p_W2, p_b2):
    src = jnp.concatenate([edge_index[0], jnp.zeros((E_PAD - E,), jnp.int32)])
    dst = jnp.concatenate([edge_index[1], jnp.full((E_PAD - E,), N, jnp.int32)])
    src = src.reshape(NS, N_WAVES, K_SLOTS, CHUNK)
    dst = dst.reshape(NS, N_WAVES, K_SLOTS, CHUNK)

    row = lambda v: v.reshape(1, -1)
    pW2 = jnp.pad(p_W2, ((0, 0), (0, HALF - O)))
    pb2 = jnp.pad(p_b2, (0, HALF - O)).reshape(1, HALF)
    batch_r = jnp.pad(batch, (0, N_PAD - N), constant_values=G).reshape(NB, 1, RB)

    xp = jnp.pad(x, ((0, N_PAD - N), (0, 0)))
    h_lo, h_hi = xp[:, :HALF], xp[:, HALF:]
    z_lo, z_hi = _sc_segsum(h_lo, h_hi, src, dst)
    h_lo, h_hi = _tc_layer(z_lo, z_hi, c0_W1, row(c0_b1), c0_W2, row(c0_b2),
                           row(ln0_g), row(ln0_b))
    z_lo, z_hi = _sc_segsum(h_lo, h_hi, src, dst)
    h_lo, h_hi = _tc_layer(z_lo, z_hi, c1_W1, row(c1_b1), c1_W2, row(c1_b2),
                           row(ln1_g), row(ln1_b))
    z_lo, z_hi = _sc_segsum(h_lo, h_hi, src, dst)
    emb, out_pad = _tc_final(z_lo, z_hi, batch_r,
                             c2_W1, row(c2_b1), c2_W2, row(c2_b2),
                             p_W1, row(p_b1), pW2, pb2)
    return (emb, out_pad[:, :O])


# speedup vs baseline: 1.1719x; 1.1267x over previous
import functools

import jax
import jax.numpy as jnp
from jax import lax
from jax.experimental import pallas as pl
from jax.experimental.pallas import tpu as pltpu
from jax.experimental.pallas import tpu_sc as plsc

N = 10000
N_PAD = 10112
D = 256
HALF = 128
E = 160000
G = 16
O = 10

NS = 16
E_PAD = 163840
EDGES_PER_TILE = E_PAD // NS
CHUNK = 128
N_CHUNKS = EDGES_PER_TILE // CHUNK
ROWS_PER_TILE = N_PAD // NS
ACC_ROWS = N_PAD

RB = 1264
NB = N_PAD // RB


K_SLOTS = 1
N_BUFS = 2
N_ISLOT = 4
N_WAVES = N_CHUNKS // K_SLOTS


@functools.partial(
    pl.kernel,
    out_type=(
        jax.ShapeDtypeStruct((N_PAD, HALF), jnp.float32),
        jax.ShapeDtypeStruct((N_PAD, HALF), jnp.float32),
    ),
    mesh=plsc.VectorSubcoreMesh(core_axis_name="c", subcore_axis_name="s"),
    scratch_types=[
        pltpu.VMEM_SHARED((ACC_ROWS, HALF), jnp.float32),
        pltpu.VMEM((N_ISLOT, K_SLOTS, CHUNK), jnp.int32),
        pltpu.VMEM((N_ISLOT, K_SLOTS, CHUNK), jnp.int32),
    ] + [pltpu.VMEM((CHUNK, HALF), jnp.float32)] * N_BUFS
      + [pltpu.SemaphoreType.DMA] * (2 * N_BUFS + 1),
)
def _sc_segsum(h_lo, h_hi, src4, dst4, z_lo, z_hi, acc, sidx, didx, *rest):
    bufs = rest[:N_BUFS]
    gsem = rest[N_BUFS:2 * N_BUFS]
    ssem = rest[2 * N_BUFS:3 * N_BUFS]
    isem = rest[3 * N_BUFS]
    c = lax.axis_index("c")
    s = lax.axis_index("s")
    row0 = s * ROWS_PER_TILE

    halves = ((0, h_lo, z_lo), (1, h_hi, z_hi))

    for cc, h_ref, _ in halves:
        @pl.when(c == cc)
        def _():
            pltpu.sync_copy(src4.at[s, 0], sidx.at[0])
            pltpu.sync_copy(dst4.at[s, 0], didx.at[0])
            pltpu.async_copy(src4.at[s, 1], sidx.at[1], isem)
            pltpu.async_copy(dst4.at[s, 1], didx.at[1], isem)
            for k in range(K_SLOTS):
                pltpu.async_copy(h_ref.at[sidx.at[0, k]], bufs[k], gsem[k])
            pltpu.sync_copy(h_ref.at[pl.ds(row0, ROWS_PER_TILE)],
                            acc.at[pl.ds(row0, ROWS_PER_TILE)])
    plsc.subcore_barrier()

    for cc, h_ref, _ in halves:
        @pl.when(c == cc)
        def _():
            def quad(q, carry):
                for u in range(4):
                    w = 4 * q + u
                    pb = u % 2
                    ob = 1 - pb
                    @pl.when(w >= 1)
                    def _():
                        pltpu.make_async_copy(
                            bufs[ob], acc.at[didx.at[0, 0]], ssem[ob]).wait()
                    @pl.when(w + 1 < N_WAVES)
                    def _():
                        pltpu.make_async_copy(
                            src4.at[s, 0], sidx.at[0], isem).wait()
                        pltpu.make_async_copy(
                            dst4.at[s, 0], didx.at[0], isem).wait()
                        pltpu.async_copy(
                            h_ref.at[sidx.at[(u + 1) % N_ISLOT, 0]],
                            bufs[ob], gsem[ob])
                    @pl.when(w + 2 < N_WAVES)
                    def _():
                        pltpu.async_copy(src4.at[s, w + 2],
                                         sidx.at[(u + 2) % N_ISLOT], isem)
                        pltpu.async_copy(dst4.at[s, w + 2],
                                         didx.at[(u + 2) % N_ISLOT], isem)
                    pltpu.make_async_copy(
                        h_ref.at[sidx.at[0, 0]], bufs[pb], gsem[pb]).wait()
                    pltpu.async_copy(bufs[pb],
                                     acc.at[didx.at[u % N_ISLOT, 0]],
                                     ssem[pb], add=True)
                return carry

            lax.fori_loop(0, N_WAVES // 4, quad, 0)
            last = (N_WAVES - 1) % 2
            pltpu.make_async_copy(bufs[last], acc.at[didx.at[0, 0]],
                                  ssem[last]).wait()
    plsc.subcore_barrier()

    for cc, _, z_ref in halves:
        @pl.when(c == cc)
        def _():
            pltpu.sync_copy(acc.at[pl.ds(row0, ROWS_PER_TILE)],
                            z_ref.at[pl.ds(row0, ROWS_PER_TILE)])


def _tc_layer_body(zl, zh, W1, b1, W2, b2, g, bln, ol, oh):
    z = jnp.concatenate([zl[...], zh[...]], axis=1)
    a = jnp.dot(z, W1[...], preferred_element_type=jnp.float32) + b1[...]
    a = jnp.maximum(a, 0.0)
    y = jnp.dot(a, W2[...], preferred_element_type=jnp.float32) + b2[...]
    y = jnp.maximum(y, 0.0)
    m = jnp.mean(y, axis=1, keepdims=True)
    v = jnp.mean((y - m) * (y - m), axis=1, keepdims=True)
    y = (y - m) * lax.rsqrt(v + 1e-5) * g[...] + bln[...]
    ol[...] = y[:, :HALF]
    oh[...] = y[:, HALF:]


def _tc_layer(zl, zh, W1, b1, W2, b2, g, bln):
    full = pl.BlockSpec((D, D), lambda i: (0, 0))
    vec = pl.BlockSpec((1, D), lambda i: (0, 0))
    half = pl.BlockSpec((RB, HALF), lambda i: (i, 0))
    return pl.pallas_call(
        _tc_layer_body,
        grid=(NB,),
        in_specs=[half, half, full, vec, full, vec, vec, vec],
        out_specs=[half, half],
        out_shape=(
            jax.ShapeDtypeStruct((N_PAD, HALF), jnp.float32),
            jax.ShapeDtypeStruct((N_PAD, HALF), jnp.float32),
        ),
    )(zl, zh, W1, b1, W2, b2, g, bln)


def _tc_final_body(zl, zh, batch_r, W1, b1, W2, b2, pW1, pb1, pW2, pb2,
                   emb_ref, out_ref, sums, cnts):
    i = pl.program_id(0)
    z = jnp.concatenate([zl[...], zh[...]], axis=1)
    a = jnp.dot(z, W1[...], preferred_element_type=jnp.float32) + b1[...]
    a = jnp.maximum(a, 0.0)
    y = jnp.dot(a, W2[...], preferred_element_type=jnp.float32) + b2[...]
    y = jnp.maximum(y, 0.0)

    b = batch_r[0, 0, :].reshape(RB, 1)
    gids = lax.broadcasted_iota(jnp.int32, (RB, G), 1)
    onehot = (b == gids).astype(jnp.float32)
    part = lax.dot_general(onehot, y, (((0,), (0,)), ((), ())),
                           preferred_element_type=jnp.float32)
    ones = jnp.ones((RB, HALF), jnp.float32)
    pcnt = lax.dot_general(onehot, ones, (((0,), (0,)), ((), ())),
                           preferred_element_type=jnp.float32)

    @pl.when(i == 0)
    def _():
        sums[...] = part
        cnts[...] = pcnt

    @pl.when(i > 0)
    def _():
        sums[...] = sums[...] + part
        cnts[...] = cnts[...] + pcnt

    @pl.when(i == NB - 1)
    def _():
        cnt = jnp.maximum(cnts[...][:, 0:1], 1.0)
        emb = sums[...] / cnt
        emb_ref[...] = emb
        z1 = jnp.dot(emb, pW1[...], preferred_element_type=jnp.float32) + pb1[...]
        z2 = jnp.dot(z1, pW2[...], preferred_element_type=jnp.float32) + pb2[...]
        mask = lax.broadcasted_iota(jnp.int32, (G, HALF), 1) < O
        zm = jnp.where(mask, z2, -1e30)
        mx = jnp.max(zm, axis=1, keepdims=True)
        se = jnp.sum(jnp.where(mask, jnp.exp(z2 - mx), 0.0), axis=1, keepdims=True)
        out_ref[...] = z2 - (jnp.log(se) + mx)


def _tc_final(zl, zh, batch_r, W1, b1, W2, b2, pW1, pb1, pW2, pb2):
    full = pl.BlockSpec((D, D), lambda i: (0, 0))
    vec = pl.BlockSpec((1, D), lambda i: (0, 0))
    half = pl.BlockSpec((RB, HALF), lambda i: (i, 0))
    return pl.pallas_call(
        _tc_final_body,
        grid=(NB,),
        in_specs=[half, half,
                  pl.BlockSpec((1, 1, RB), lambda i: (i, 0, 0)),
                  full, vec, full, vec,
                  full, vec,
                  pl.BlockSpec((D, HALF), lambda i: (0, 0)),
                  pl.BlockSpec((1, HALF), lambda i: (0, 0))],
        out_specs=[pl.BlockSpec((G, D), lambda i: (0, 0)),
                   pl.BlockSpec((G, HALF), lambda i: (0, 0))],
        out_shape=(
            jax.ShapeDtypeStruct((G, D), jnp.float32),
            jax.ShapeDtypeStruct((G, HALF), jnp.float32),
        ),
        scratch_shapes=[pltpu.VMEM((G, D), jnp.float32),
                        pltpu.VMEM((G, HALF), jnp.float32)],
    )(zl, zh, batch_r, W1, b1, W2, b2, pW1, pb1, pW2, pb2)


def kernel(x, edge_index, batch,
           c0_W1, c0_b1, c0_W2, c0_b2,
           c1_W1, c1_b1, c1_W2, c1_b2,
           c2_W1, c2_b1, c2_W2, c2_b2,
           ln0_g, ln0_b, ln1_g, ln1_b,
           p_W1, p_b1, p_W2, p_b2):
    src = jnp.concatenate([edge_index[0], jnp.zeros((E_PAD - E,), jnp.int32)])
    dst = jnp.concatenate([edge_index[1], jnp.full((E_PAD - E,), N, jnp.int32)])
    src = src.reshape(NS, N_WAVES, K_SLOTS, CHUNK)
    dst = dst.reshape(NS, N_WAVES, K_SLOTS, CHUNK)

    row = lambda v: v.reshape(1, -1)
    pW2 = jnp.pad(p_W2, ((0, 0), (0, HALF - O)))
    pb2 = jnp.pad(p_b2, (0, HALF - O)).reshape(1, HALF)
    batch_r = jnp.pad(batch, (0, N_PAD - N), constant_values=G).reshape(NB, 1, RB)

    xp = jnp.pad(x, ((0, N_PAD - N), (0, 0)))
    h_lo, h_hi = xp[:, :HALF], xp[:, HALF:]
    z_lo, z_hi = _sc_segsum(h_lo, h_hi, src, dst)
    h_lo, h_hi = _tc_layer(z_lo, z_hi, c0_W1, row(c0_b1), c0_W2, row(c0_b2),
                           row(ln0_g), row(ln0_b))
    z_lo, z_hi = _sc_segsum(h_lo, h_hi, src, dst)
    h_lo, h_hi = _tc_layer(z_lo, z_hi, c1_W1, row(c1_b1), c1_W2, row(c1_b2),
                           row(ln1_g), row(ln1_b))
    z_lo, z_hi = _sc_segsum(h_lo, h_hi, src, dst)
    emb, out_pad = _tc_final(z_lo, z_hi, batch_r,
                             c2_W1, row(c2_b1), c2_W2, row(c2_b2),
                             p_W1, row(p_b1), pW2, pb2)
    return (emb, out_pad[:, :O])
